# baseline (speedup 1.0000x reference)
import functools

import jax
import jax.numpy as jnp
from jax import lax
from jax.experimental import pallas as pl
from jax.experimental.pallas import tpu as pltpu
from jax.experimental.pallas import tpu_sc as plsc

DIM = 64
BATCH = 16384
REL_ROWS = 1000
NC = 2
NS = 16
NW = NC * NS
BPW = BATCH // NW
C = 128
NCH = BPW // C
G = C // 16


def _transe_body(bh, bt, br, ent, rel1d, out_hbm,
                 idx_h, idx_t, idx_r, hv, tv, relv, ov, sem0, sem1):
    wid = lax.axis_index("s") * NC + lax.axis_index("c")
    base = wid * BPW
    sems = (sem0, sem1)

    pltpu.sync_copy(bh.at[pl.ds(base, BPW)], idx_h)
    pltpu.sync_copy(bt.at[pl.ds(base, BPW)], idx_t)
    pltpu.sync_copy(br.at[pl.ds(base, BPW)], idx_r)
    pltpu.sync_copy(rel1d, relv)

    lanes = lax.iota(jnp.int32, 16)
    perms = [lanes ^ (1 << b) for b in range(4)]
    dn = lax.GatherDimensionNumbers(
        offset_dims=(), collapsed_slice_dims=(0,), start_index_map=(0,))

    def lane_sum(s):
        for p in perms:
            s = s + lax.gather(s, p[:, None], dn, (1,),
                               mode=lax.GatherScatterMode.PROMISE_IN_BOUNDS)
        return s

    def fire(ch, sem):
        b = ch & 1

        @pl.loop(0, G)
        def _fire(g):
            jh = idx_h[pl.ds(ch * C + g * 16, 16)]
            jt = idx_t[pl.ds(ch * C + g * 16, 16)]
            for k in range(16):
                dst = b * C + g * 16 + k
                pltpu.async_copy(ent.at[pl.ds(jh[k], 1)],
                                 hv.at[pl.ds(dst, 1)], sem)
                pltpu.async_copy(ent.at[pl.ds(jt[k], 1)],
                                 tv.at[pl.ds(dst, 1)], sem)

    def drain(ch, sem):
        b = ch & 1

        @pl.loop(0, C, unroll=8)
        def _drain(i):
            dst = b * C + i
            pltpu.make_async_copy(ent.at[pl.ds(0, 1)],
                                  hv.at[pl.ds(dst, 1)], sem).wait()
            pltpu.make_async_copy(ent.at[pl.ds(0, 1)],
                                  tv.at[pl.ds(dst, 1)], sem).wait()

    def _unpack16(words):
        return plsc.unpack(plsc.bitcast(words, jnp.bfloat16),
                           format=plsc.PackFormat.INTERLEAVED)

    def l1_terms(hrow, trow, rbase, half):
        hx = _unpack16(hrow)
        tx = _unpack16(trow)
        rx = _unpack16(relv[pl.ds(rbase + half * 16, 16)])
        return (jnp.abs(hx[0] + rx[0] - tx[0])
                + jnp.abs(hx[1] + rx[1] - tx[1]))

    def compute(ch):
        b = ch & 1

        def group_body(g, _):
            jrv = idx_r[pl.ds(ch * C + g * 16, 16)] * (DIM // 2)
            acc = jnp.zeros((16,), jnp.float32)
            for k in range(16):
                i = b * C + g * 16 + k
                rbase = jrv[k]
                s = (l1_terms(hv[i, pl.ds(0, 16)], tv[i, pl.ds(0, 16)],
                              rbase, 0)
                     + l1_terms(hv[i, pl.ds(16, 16)], tv[i, pl.ds(16, 16)],
                                rbase, 1))
                acc = jnp.where(lanes == k, lane_sum(s), acc)
            ov[pl.ds(ch * C + g * 16, 16)] = acc
            return 0

        lax.fori_loop(0, G, group_body, 0)

    fire(0, sems[0])
    for ch in range(NCH):
        if ch + 1 < NCH:
            fire(ch + 1, sems[(ch + 1) & 1])
        drain(ch, sems[ch & 1])
        compute(ch)

    pltpu.sync_copy(ov, out_hbm.at[pl.ds(base, BPW)])


_transe = functools.partial(
    pl.kernel,
    out_type=jax.ShapeDtypeStruct((BATCH,), jnp.float32),
    mesh=plsc.VectorSubcoreMesh(core_axis_name="c", subcore_axis_name="s"),
    scratch_types=[
        pltpu.VMEM((BPW,), jnp.int32),
        pltpu.VMEM((BPW,), jnp.int32),
        pltpu.VMEM((BPW,), jnp.int32),
        pltpu.VMEM((2 * C, DIM // 2), jnp.int32),
        pltpu.VMEM((2 * C, DIM // 2), jnp.int32),
        pltpu.VMEM((REL_ROWS * DIM // 2,), jnp.int32),
        pltpu.VMEM((BPW,), jnp.float32),
        pltpu.SemaphoreType.DMA,
        pltpu.SemaphoreType.DMA,
    ],
    compiler_params=pltpu.CompilerParams(needs_layout_passes=False),
)(_transe_body)


@jax.jit
def kernel(batch_h, batch_t, batch_r, ent_emb, rel_emb):
    def pack_bf16_pairs(table):
        xi = lax.bitcast_convert_type(table, jnp.uint32)
        b = (xi + 0x7FFF + ((xi >> 16) & 1)) >> 16
        e = b[:, 0::2]
        o = b[:, 1::2]
        return lax.bitcast_convert_type(e | (o << 16), jnp.int32)

    ent16 = pack_bf16_pairs(ent_emb)
    rel16 = pack_bf16_pairs(rel_emb).reshape(-1)
    return _transe(batch_h, batch_t, batch_r, ent16, rel16)


# speedup vs baseline: 23.8981x; 23.8981x over previous
import functools

import jax
import jax.numpy as jnp
from jax import lax
from jax.experimental import pallas as pl
from jax.experimental.pallas import tpu as pltpu
from jax.experimental.pallas import tpu_sc as plsc

DIM = 64
BATCH = 16384
REL_ROWS = 1000
NC = 2
NS = 16
NW = NC * NS
BPW = BATCH // NW
C = 64
NCH = BPW // C
G = C // 16


def _transe_body(bh, bt, br, ent, rel1d, out_hbm,
                 idx_h, idx_t, idx_r, hv, tv, relv, ov, sem0, sem1):
    wid = lax.axis_index("s") * NC + lax.axis_index("c")
    base = wid * BPW
    sems = (sem0, sem1)

    pltpu.sync_copy(bh.at[pl.ds(base, BPW)], idx_h)
    pltpu.sync_copy(bt.at[pl.ds(base, BPW)], idx_t)
    pltpu.sync_copy(br.at[pl.ds(base, BPW)], idx_r)
    pltpu.sync_copy(rel1d, relv)

    lanes = lax.iota(jnp.int32, 16)

    def fire(ch, sem):
        b = ch & 1

        @pl.loop(0, G)
        def _fire(g):
            jh = idx_h[pl.ds(ch * C + g * 16, 16)]
            jt = idx_t[pl.ds(ch * C + g * 16, 16)]
            for k in range(16):
                dst = b * C + g * 16 + k
                pltpu.async_copy(ent.at[pl.ds(jh[k], 1)],
                                 hv.at[pl.ds(dst, 1)], sem)
                pltpu.async_copy(ent.at[pl.ds(jt[k], 1)],
                                 tv.at[pl.ds(dst, 1)], sem)

    def drain(ch, sem):
        b = ch & 1

        @pl.loop(0, C, unroll=8)
        def _drain(i):
            dst = b * C + i
            pltpu.make_async_copy(ent.at[pl.ds(0, 1)],
                                  hv.at[pl.ds(dst, 1)], sem).wait()
            pltpu.make_async_copy(ent.at[pl.ds(0, 1)],
                                  tv.at[pl.ds(dst, 1)], sem).wait()

    def compute(ch):
        b = ch & 1

        def group_body(g, _):
            rows = b * C + g * 16 + lanes
            jr = idx_r[pl.ds(ch * C + g * 16, 16)] * DIM

            def col_body(j, acc):
                colj = jnp.full((16,), 0, jnp.int32) + j
                hg = plsc.load_gather(hv, [rows, colj])
                tg = plsc.load_gather(tv, [rows, colj])
                rg = plsc.load_gather(relv, [jr + j])
                return acc + jnp.abs(hg + rg - tg)

            acc = lax.fori_loop(0, DIM, col_body, jnp.zeros((16,), jnp.float32))
            ov[pl.ds(ch * C + g * 16, 16)] = acc
            return 0

        lax.fori_loop(0, G, group_body, 0)

    fire(0, sems[0])
    for ch in range(NCH):
        if ch + 1 < NCH:
            fire(ch + 1, sems[(ch + 1) & 1])
        drain(ch, sems[ch & 1])
        compute(ch)

    pltpu.sync_copy(ov, out_hbm.at[pl.ds(base, BPW)])


_transe = functools.partial(
    pl.kernel,
    out_type=jax.ShapeDtypeStruct((BATCH,), jnp.float32),
    mesh=plsc.VectorSubcoreMesh(core_axis_name="c", subcore_axis_name="s"),
    scratch_types=[
        pltpu.VMEM((BPW,), jnp.int32),
        pltpu.VMEM((BPW,), jnp.int32),
        pltpu.VMEM((BPW,), jnp.int32),
        pltpu.VMEM((2 * C, DIM), jnp.float32),
        pltpu.VMEM((2 * C, DIM), jnp.float32),
        pltpu.VMEM((REL_ROWS * DIM,), jnp.float32),
        pltpu.VMEM((BPW,), jnp.float32),
        pltpu.SemaphoreType.DMA,
        pltpu.SemaphoreType.DMA,
    ],
    compiler_params=pltpu.CompilerParams(use_tc_tiling_on_sc=True, needs_layout_passes=False),
)(_transe_body)


@jax.jit
def kernel(batch_h, batch_t, batch_r, ent_emb, rel_emb):
    return _transe(batch_h, batch_t, batch_r, ent_emb, rel_emb.reshape(-1))


# speedup vs baseline: 26.6713x; 1.1160x over previous
import functools

import jax
import jax.numpy as jnp
from jax import lax
from jax.experimental import pallas as pl
from jax.experimental.pallas import tpu as pltpu
from jax.experimental.pallas import tpu_sc as plsc

DIM = 64
BATCH = 16384
NC = 2
NS = 16
NW = NC * NS
BPW = BATCH // NW
C = 128
NCH = BPW // C


def _transe_body(bh, bt, br, ent, rel, out_hbm,
                 idx_h, idx_t, idx_r, hv, tv, rv, ov, sem):
    wid = lax.axis_index("s") * NC + lax.axis_index("c")
    base = wid * BPW

    pltpu.sync_copy(bh.at[pl.ds(base, BPW)], idx_h)
    pltpu.sync_copy(bt.at[pl.ds(base, BPW)], idx_t)
    pltpu.sync_copy(br.at[pl.ds(base, BPW)], idx_r)

    lanes = lax.iota(jnp.int32, 16)
    perms = [lanes ^ (1 << b) for b in range(4)]
    dn = lax.GatherDimensionNumbers(
        offset_dims=(), collapsed_slice_dims=(0,), start_index_map=(0,))

    def shuffle(x, idx):
        return lax.gather(x, idx[:, None], dn, (1,),
                          mode=lax.GatherScatterMode.PROMISE_IN_BOUNDS)

    def lane_sum(s):
        for p in perms:
            s = s + shuffle(s, p)
        return s

    for ch in range(NCH):
        @pl.loop(0, C // 16)
        def _fire(g):
            jh = idx_h[pl.ds(ch * C + g * 16, 16)]
            jt = idx_t[pl.ds(ch * C + g * 16, 16)]
            jr = idx_r[pl.ds(ch * C + g * 16, 16)]
            for k in range(16):
                pltpu.async_copy(ent.at[pl.ds(jh[k], 1)],
                                 hv.at[pl.ds(g * 16 + k, 1)], sem)
                pltpu.async_copy(ent.at[pl.ds(jt[k], 1)],
                                 tv.at[pl.ds(g * 16 + k, 1)], sem)
                pltpu.async_copy(rel.at[pl.ds(jr[k], 1)],
                                 rv.at[pl.ds(g * 16 + k, 1)], sem)

        @pl.loop(0, C, unroll=8)
        def _drain(i):
            pltpu.make_async_copy(ent.at[pl.ds(0, 1)],
                                  hv.at[pl.ds(i, 1)], sem).wait()
            pltpu.make_async_copy(ent.at[pl.ds(0, 1)],
                                  tv.at[pl.ds(i, 1)], sem).wait()
            pltpu.make_async_copy(rel.at[pl.ds(0, 1)],
                                  rv.at[pl.ds(i, 1)], sem).wait()

        def group_body(g, _):
            acc = jnp.zeros((16,), jnp.float32)
            for k in range(16):
                i = g * 16 + k
                s = None
                for c in range(DIM // 16):
                    a = jnp.abs(hv[i, pl.ds(c * 16, 16)]
                                + rv[i, pl.ds(c * 16, 16)]
                                - tv[i, pl.ds(c * 16, 16)])
                    s = a if s is None else s + a
                acc = jnp.where(lanes == k, lane_sum(s), acc)
            ov[pl.ds(ch * C + g * 16, 16)] = acc
            return 0

        lax.fori_loop(0, C // 16, group_body, 0)

    pltpu.sync_copy(ov, out_hbm.at[pl.ds(base, BPW)])


_transe = functools.partial(
    pl.kernel,
    out_type=jax.ShapeDtypeStruct((BATCH,), jnp.float32),
    mesh=plsc.VectorSubcoreMesh(core_axis_name="c", subcore_axis_name="s"),
    scratch_types=[
        pltpu.VMEM((BPW,), jnp.int32),
        pltpu.VMEM((BPW,), jnp.int32),
        pltpu.VMEM((BPW,), jnp.int32),
        pltpu.VMEM((C, DIM), jnp.float32),
        pltpu.VMEM((C, DIM), jnp.float32),
        pltpu.VMEM((C, DIM), jnp.float32),
        pltpu.VMEM((BPW,), jnp.float32),
        pltpu.SemaphoreType.DMA,
    ],
)(_transe_body)


@jax.jit
def kernel(batch_h, batch_t, batch_r, ent_emb, rel_emb):
    return _transe(batch_h, batch_t, batch_r, ent_emb, rel_emb)
